# Initial kernel scaffold; baseline (speedup 1.0000x reference)
import jax
import jax.numpy as jnp
from jax.experimental import pallas as pl


def kernel(q, k, v, means):
    raise NotImplementedError("write your pallas kernel here")


# speedup vs baseline: 3.1688x; 3.1688x over previous
import functools

import jax
import jax.numpy as jnp
from jax.experimental import pallas as pl
from jax.experimental.pallas import tpu as pltpu

NUM_CLUSTERS = 32
WINDOW_SIZE = 128
COMMITMENT = 1e-4

INTERPRET = False


def _dists_body(q_ref, k_ref, m_ref, dq_ref, dk_ref, aux_ref):
    m = m_ref[0]
    msq = jnp.sum(m * m, axis=1)

    def stats(x):
        nrm = jnp.sqrt(jnp.sum(x * x, axis=1, keepdims=True))
        xn = x / jnp.maximum(nrm, 1e-12)
        d = jax.lax.dot_general(m, xn, (((1,), (1,)), ((), ())),
                                preferred_element_type=jnp.float32)
        s = jnp.sum(xn * xn, axis=1)
        dmax = jnp.max(d, axis=0)
        amax = jnp.argmax(d, axis=0)
        sel = jax.lax.broadcasted_iota(jnp.int32, d.shape, 0) == amax[None, :]
        msqsel = jnp.sum(jnp.where(sel, msq[:, None], 0.0), axis=0)
        part = jnp.sum(s - 2.0 * dmax + msqsel)
        return d, part

    dq, pq = stats(q_ref[0, 0])
    dk, pk = stats(k_ref[0, 0])
    dq_ref[0, 0] = dq
    dk_ref[0, 0] = dk
    r = jax.lax.broadcasted_iota(jnp.int32, (8, 128), 0)
    c = jax.lax.broadcasted_iota(jnp.int32, (8, 128), 1)
    aux_ref[0, 0] = jnp.where((r == 0) & (c == 0), pq + pk, 0.0)


def _dists_call(q, k, means):
    b, h, t, d = q.shape
    nc = means.shape[1]
    grid = (b, h)
    return pl.pallas_call(
        _dists_body,
        grid=grid,
        in_specs=[
            pl.BlockSpec((1, 1, t, d), lambda i, j: (i, j, 0, 0)),
            pl.BlockSpec((1, 1, t, d), lambda i, j: (i, j, 0, 0)),
            pl.BlockSpec((1, nc, d), lambda i, j: (j, 0, 0)),
        ],
        out_specs=[
            pl.BlockSpec((1, 1, nc, t), lambda i, j: (i, j, 0, 0)),
            pl.BlockSpec((1, 1, nc, t), lambda i, j: (i, j, 0, 0)),
            pl.BlockSpec((1, 1, 8, 128), lambda i, j: (i, j, 0, 0)),
        ],
        out_shape=[
            jax.ShapeDtypeStruct((b, h, nc, t), jnp.float32),
            jax.ShapeDtypeStruct((b, h, nc, t), jnp.float32),
            jax.ShapeDtypeStruct((b, h, 8, 128), jnp.float32),
        ],
        interpret=INTERPRET,
    )(q, k, means)


def _attn_body(qg_ref, kg_ref, vg_ref, o_ref, *, nc, wsz, scale):
    for c in range(nc):
        sl = slice(c * wsz, (c + 1) * wsz)
        qc = qg_ref[0, sl, :]
        kc = kg_ref[0, sl, :]
        vc = vg_ref[0, sl, :]
        dots = jax.lax.dot_general(qc, kc, (((1,), (1,)), ((), ())),
                                   preferred_element_type=jnp.float32) * scale
        mx = jnp.max(dots, axis=1, keepdims=True)
        e = jnp.exp(dots - mx)
        p = e / jnp.sum(e, axis=1, keepdims=True)
        o_ref[0, sl, :] = jnp.dot(p, vc, preferred_element_type=jnp.float32)


def _attn_call(qg, kg, vg):
    bh, n, d = qg.shape
    nc = NUM_CLUSTERS
    wsz = n // nc
    body = functools.partial(_attn_body, nc=nc, wsz=wsz, scale=d ** -0.5)
    return pl.pallas_call(
        body,
        grid=(bh,),
        in_specs=[pl.BlockSpec((1, n, d), lambda i: (i, 0, 0))] * 3,
        out_specs=pl.BlockSpec((1, n, d), lambda i: (i, 0, 0)),
        out_shape=jax.ShapeDtypeStruct((bh, n, d), jnp.float32),
        interpret=INTERPRET,
    )(qg, kg, vg)


def kernel(q, k, v, means):
    b, h, t, d = q.shape
    nc = NUM_CLUSTERS
    wsz = min(WINDOW_SIZE, t)

    dq, dk, aux_parts = _dists_call(q, k, means)
    aux_loss = jnp.sum(aux_parts) * (COMMITMENT / (b * h * 2 * t * d))

    _, idx_q = jax.lax.top_k(dq, wsz)
    _, idx_k = jax.lax.top_k(dk, wsz)

    iq = idx_q.reshape(b, h, nc * wsz)
    ik = idx_k.reshape(b, h, nc * wsz)
    qg = jnp.take_along_axis(q, iq[..., None], axis=2)
    kg = jnp.take_along_axis(k, ik[..., None], axis=2)
    vg = jnp.take_along_axis(v, ik[..., None], axis=2)

    so = _attn_call(qg.reshape(b * h, nc * wsz, d),
                    kg.reshape(b * h, nc * wsz, d),
                    vg.reshape(b * h, nc * wsz, d)).reshape(b, h, nc * wsz, d)

    def _one(t_bh, idx_bh):
        z = jnp.zeros((t, d), jnp.float32)
        numer = z.at[idx_bh].add(t_bh)
        denom = z.at[idx_bh].add(jnp.ones_like(t_bh))
        return numer / (denom + 1e-5)

    out = jax.vmap(jax.vmap(_one))(so, iq)
    return out, aux_loss
